# baseline (speedup 1.0000x reference)
import functools

import jax
import jax.numpy as jnp
from jax import lax
from jax.experimental import pallas as pl
from jax.experimental.pallas import tpu as pltpu
from jax.experimental.pallas import tpu_sc as plsc

_D = 64
_NC = 2
_NS = 16
_NW = _NC * _NS
_C = 640
_G = 128
_NB = 2
_BB = 512
_K = 2


@functools.partial(jax.jit, static_argnums=(2, 3))
def _sc_gather(idx_flat, table, rows, row0):
    b_per_w = rows // _NW
    n_chunks = b_per_w // _C

    @functools.partial(
        pl.kernel,
        out_type=jax.ShapeDtypeStruct((rows, _D), jnp.float32),
        mesh=plsc.VectorSubcoreMesh(core_axis_name="c", subcore_axis_name="s"),
        scratch_types=[
            pltpu.VMEM((b_per_w,), jnp.int32),
            pltpu.VMEM((_NB, _C, _D), jnp.float32),
            pltpu.SemaphoreType.DMA,
            pltpu.SemaphoreType.DMA,
        ],
        compiler_params=pltpu.CompilerParams(use_tc_tiling_on_sc=False),
    )
    def body(idx_hbm, table_hbm, out_hbm, idx_v, rows_v, gsem, ssem):
        wid = lax.axis_index("s") * _NC + lax.axis_index("c")
        base = wid * b_per_w
        pltpu.sync_copy(idx_hbm.at[pl.ds(row0 + base, b_per_w)], idx_v)

        def issue_gathers(g, bf):
            ioff = pl.multiple_of(g * _C, 8)
            return [
                pltpu.async_copy(
                    table_hbm.at[idx_v.at[pl.ds(ioff + j * _G, _G)]],
                    rows_v.at[bf].at[pl.ds(j * _G, _G)],
                    gsem,
                )
                for j in range(_C // _G)
            ]

        def issue_store(g, bf):
            off = pl.multiple_of(base + g * _C, 8)
            pltpu.async_copy(rows_v.at[bf], out_hbm.at[pl.ds(off, _C)], ssem)

        def drain_store():
            pltpu.make_async_copy(
                rows_v.at[0], out_hbm.at[pl.ds(base, _C)], ssem
            ).wait()

        for bf in range(_NB):
            for cp in issue_gathers(bf, bf):
                cp.wait()
            issue_store(bf, bf)

        def steady(t, carry):
            for bf in range(_NB):
                g = _NB * t + bf
                drain_store()
                for cp in issue_gathers(g, bf):
                    cp.wait()
                issue_store(g, bf)
            return carry

        lax.fori_loop(1, n_chunks // _NB, steady, 0)

        for _ in range(_NB):
            drain_store()

    return body(idx_flat, table)


def _tc_transpose_chunk(a, acc, b, hd, col_block0, nsteps):
    m = hd // 128
    rows_per_step = _BB * m

    if acc is None:
        def body(a_ref, o_ref):
            x = a_ref[...]
            x3 = x.reshape(_BB, m, 128)
            for j in range(m):
                o_ref[pl.ds(j * 128, 128), :] = jnp.transpose(x3[:, j, :], (1, 0))

        return pl.pallas_call(
            body,
            grid=(nsteps,),
            in_specs=[pl.BlockSpec((rows_per_step, 128), lambda i: (i, 0))],
            out_specs=pl.BlockSpec(
                (hd, _BB), lambda i, o=col_block0: (0, o + i)
            ),
            out_shape=jax.ShapeDtypeStruct((hd, b), jnp.float32),
        )(a)

    def body2(a_ref, acc_ref, o_ref):
        x = a_ref[...]
        x3 = x.reshape(_BB, m, 128)
        for j in range(m):
            o_ref[pl.ds(j * 128, 128), :] = jnp.transpose(x3[:, j, :], (1, 0))

    return pl.pallas_call(
        body2,
        grid=(nsteps,),
        in_specs=[
            pl.BlockSpec((rows_per_step, 128), lambda i: (i, 0)),
            pl.BlockSpec(memory_space=pl.ANY),
        ],
        out_specs=pl.BlockSpec((hd, _BB), lambda i, o=col_block0: (0, o + i)),
        out_shape=jax.ShapeDtypeStruct((hd, b), jnp.float32),
        input_output_aliases={1: 0},
    )(a, acc)


def kernel(timesteps, pe):
    b, h = timesteps.shape
    d = pe.shape[1]
    hd = h * d
    flat = timesteps.reshape(-1)
    total = b * h
    rows_k = total // _K
    bk = b // _K
    nsteps = bk // _BB

    gathered = [
        _sc_gather(flat, pe, rows_k, k * rows_k) for k in range(_K)
    ]
    acc = None
    for k in range(_K):
        a_k = gathered[k].reshape(rows_k * d // 128, 128)
        acc = _tc_transpose_chunk(
            a_k, acc, b, hd, k * nsteps, nsteps
        )
    return jnp.transpose(acc.reshape(h, d, b), (2, 0, 1))


# speedup vs baseline: 1.0044x; 1.0044x over previous
import functools

import jax
import jax.numpy as jnp
from jax import lax
from jax.experimental import pallas as pl
from jax.experimental.pallas import tpu as pltpu
from jax.experimental.pallas import tpu_sc as plsc

_D = 64
_NC = 2
_NS = 16
_NW = _NC * _NS
_C = 640
_G = 128
_NB = 2
_BB = 512
_K = 4


@functools.partial(jax.jit, static_argnums=(2, 3))
def _sc_gather(idx_flat, table, rows, row0):
    b_per_w = rows // _NW
    n_chunks = b_per_w // _C

    @functools.partial(
        pl.kernel,
        out_type=jax.ShapeDtypeStruct((rows, _D), jnp.float32),
        mesh=plsc.VectorSubcoreMesh(core_axis_name="c", subcore_axis_name="s"),
        scratch_types=[
            pltpu.VMEM((b_per_w,), jnp.int32),
            pltpu.VMEM((_NB, _C, _D), jnp.float32),
            pltpu.SemaphoreType.DMA,
            pltpu.SemaphoreType.DMA,
        ],
        compiler_params=pltpu.CompilerParams(use_tc_tiling_on_sc=False),
    )
    def body(idx_hbm, table_hbm, out_hbm, idx_v, rows_v, gsem, ssem):
        wid = lax.axis_index("s") * _NC + lax.axis_index("c")
        base = wid * b_per_w
        pltpu.sync_copy(idx_hbm.at[pl.ds(row0 + base, b_per_w)], idx_v)

        def issue_gathers(g, bf):
            ioff = pl.multiple_of(g * _C, 8)
            return [
                pltpu.async_copy(
                    table_hbm.at[idx_v.at[pl.ds(ioff + j * _G, _G)]],
                    rows_v.at[bf].at[pl.ds(j * _G, _G)],
                    gsem,
                )
                for j in range(_C // _G)
            ]

        def issue_store(g, bf):
            off = pl.multiple_of(base + g * _C, 8)
            pltpu.async_copy(rows_v.at[bf], out_hbm.at[pl.ds(off, _C)], ssem)

        def drain_store():
            pltpu.make_async_copy(
                rows_v.at[0], out_hbm.at[pl.ds(base, _C)], ssem
            ).wait()

        for bf in range(_NB):
            for cp in issue_gathers(bf, bf):
                cp.wait()
            issue_store(bf, bf)

        def steady(t, carry):
            for bf in range(_NB):
                g = _NB * t + bf
                drain_store()
                for cp in issue_gathers(g, bf):
                    cp.wait()
                issue_store(g, bf)
            return carry

        lax.fori_loop(1, n_chunks // _NB, steady, 0)

        for _ in range(_NB):
            drain_store()

    return body(idx_flat, table)


def _tc_transpose_chunk(a, acc, b, hd, col_block0, nsteps):
    m = hd // 128
    rows_per_step = _BB * m

    if acc is None:
        def body(a_ref, o_ref):
            x = a_ref[...]
            x3 = x.reshape(_BB, m, 128)
            for j in range(m):
                o_ref[pl.ds(j * 128, 128), :] = jnp.transpose(x3[:, j, :], (1, 0))

        return pl.pallas_call(
            body,
            grid=(nsteps,),
            in_specs=[pl.BlockSpec((rows_per_step, 128), lambda i: (i, 0))],
            out_specs=pl.BlockSpec(
                (hd, _BB), lambda i, o=col_block0: (0, o + i)
            ),
            out_shape=jax.ShapeDtypeStruct((hd, b), jnp.float32),
        )(a)

    def body2(a_ref, acc_ref, o_ref):
        x = a_ref[...]
        x3 = x.reshape(_BB, m, 128)
        for j in range(m):
            o_ref[pl.ds(j * 128, 128), :] = jnp.transpose(x3[:, j, :], (1, 0))

    return pl.pallas_call(
        body2,
        grid=(nsteps,),
        in_specs=[
            pl.BlockSpec((rows_per_step, 128), lambda i: (i, 0)),
            pl.BlockSpec(memory_space=pl.ANY),
        ],
        out_specs=pl.BlockSpec((hd, _BB), lambda i, o=col_block0: (0, o + i)),
        out_shape=jax.ShapeDtypeStruct((hd, b), jnp.float32),
        input_output_aliases={1: 0},
    )(a, acc)


def kernel(timesteps, pe):
    b, h = timesteps.shape
    d = pe.shape[1]
    hd = h * d
    flat = timesteps.reshape(-1)
    total = b * h
    rows_k = total // _K
    bk = b // _K
    nsteps = bk // _BB

    gathered = [
        _sc_gather(flat, pe, rows_k, k * rows_k) for k in range(_K)
    ]
    acc = None
    for k in range(_K):
        a_k = gathered[k].reshape(rows_k * d // 128, 128)
        acc = _tc_transpose_chunk(
            a_k, acc, b, hd, k * nsteps, nsteps
        )
    return jnp.transpose(acc.reshape(h, d, b), (2, 0, 1))
